# Initial kernel scaffold; baseline (speedup 1.0000x reference)
import jax
import jax.numpy as jnp
from jax.experimental import pallas as pl


def kernel(images, labels, We1, be1, We2, be2, We3, be3, Wd1, bd1, Wd2, bd2, Wd3, bd3, a, b, eps):
    raise NotImplementedError("write your pallas kernel here")


# speedup vs baseline: 2.5409x; 2.5409x over previous
import functools

import jax
import jax.numpy as jnp
from jax import lax
from jax.experimental import pallas as pl

LATENT_DIM = 128
OUT_DIM = 512
IN_DIM = 1024
N_CLASSES = 10
N_HASHES = 16
NUM_BUCKETS = 1024
W_BUCKET = 4.0
BATCH = 4096
TILE = 512
GRID = BATCH // TILE
_SOFTPLUS_INV_1 = 0.5413248546129181
_PRIMES_LIST = [3, 5, 7, 11, 13, 17, 19, 23, 29, 31, 37, 41, 43, 47, 53, 59]

_f32 = jnp.float32


def _fwd_body(x_ref, lab_ref, eps_ref, We1_ref, be1_ref, We2_ref, be2_ref,
              We3_ref, be3_ref, Wd1z_ref, Wd1l_ref, bd1_ref, Wd2_ref, bd2_ref,
              Wd3_ref, bd3_ref, a_ref, b_ref, primes_ref,
              out_ref, bucket_ref, sums_ref, counts_ref):
    i = pl.program_id(0)
    x = x_ref[...]
    h = jnp.maximum(jnp.dot(x, We1_ref[...], preferred_element_type=_f32) + be1_ref[...], 0.0)
    h = jnp.maximum(jnp.dot(h, We2_ref[...], preferred_element_type=_f32) + be2_ref[...], 0.0)
    ts = jnp.dot(h, We3_ref[...], preferred_element_type=_f32) + be3_ref[...]
    loc = ts[:, :LATENT_DIM]
    raw = ts[:, LATENT_DIM:] + _SOFTPLUS_INV_1
    scale = jnp.maximum(raw, 0.0) + jnp.log(1.0 + jnp.exp(-jnp.abs(raw)))
    z = loc + scale * eps_ref[...]
    pre = (jnp.dot(z, Wd1z_ref[...], preferred_element_type=_f32)
           + jnp.dot(lab_ref[...], Wd1l_ref[...], preferred_element_type=_f32)
           + bd1_ref[...])
    h = jnp.maximum(pre, 0.0)
    h = jnp.maximum(jnp.dot(h, Wd2_ref[...], preferred_element_type=_f32) + bd2_ref[...], 0.0)
    out = jnp.dot(h, Wd3_ref[...], preferred_element_type=_f32) + bd3_ref[...]
    out_ref[...] = out

    hv = jnp.dot(out, a_ref[...], preferred_element_type=_f32) + b_ref[...]
    codes = jnp.floor(hv * (1.0 / W_BUCKET)).astype(jnp.int32)
    c2 = (codes * primes_ref[...]) & (NUM_BUCKETS - 1)
    bucket = jnp.sum(c2, axis=1, keepdims=True) & (NUM_BUCKETS - 1)
    bucket_ref[...] = bucket

    onehot = (bucket == lax.broadcasted_iota(jnp.int32, (TILE, NUM_BUCKETS), 1)).astype(_f32)
    contrib = lax.dot_general(onehot, out, (((0,), (0,)), ((), ())),
                              preferred_element_type=_f32,
                              precision=lax.Precision.HIGHEST)
    ones_col = jnp.ones((TILE, 1), _f32)
    cnt = lax.dot_general(onehot, ones_col, (((0,), (0,)), ((), ())),
                          preferred_element_type=_f32)

    @pl.when(i == 0)
    def _():
        sums_ref[...] = jnp.zeros_like(sums_ref)
        counts_ref[...] = jnp.zeros_like(counts_ref)

    sums_ref[...] += contrib
    counts_ref[...] += cnt


def _q_body(out_ref, sums_ref, counts_ref, q_ref):
    out = out_ref[...]
    means = sums_ref[...] / jnp.maximum(counts_ref[...], 1.0)
    m2_row = jnp.sum(means * means, axis=1).reshape(1, NUM_BUCKETS)
    g = lax.dot_general(out, means, (((1,), (1,)), ((), ())),
                        preferred_element_type=_f32)
    rowsq = jnp.sum(out * out, axis=1, keepdims=True)
    d2 = rowsq + m2_row - 2.0 * g
    d2 = jnp.maximum(d2, 0.0)
    qraw = 1.0 / (1.0 + d2)
    q_ref[...] = qraw / jnp.sum(qraw, axis=1, keepdims=True)


@functools.partial(jax.jit, static_argnames=("interpret",))
def _run(images, labels, We1, be1, We2, be2, We3, be3,
         Wd1, bd1, Wd2, bd2, Wd3, bd3, a, b, eps, interpret=False):
    primes = jnp.array([_PRIMES_LIST], jnp.int32)
    be1r = be1.reshape(1, -1)
    be2r = be2.reshape(1, -1)
    be3r = be3.reshape(1, -1)
    bd1r = bd1.reshape(1, -1)
    bd2r = bd2.reshape(1, -1)
    bd3r = bd3.reshape(1, -1)
    br = b.reshape(1, -1)
    Wd1z = Wd1[:LATENT_DIM]
    Wd1l = Wd1[LATENT_DIM:]

    full = lambda shape: pl.BlockSpec(shape, lambda i: (0,) * len(shape))
    tiled = lambda shape: pl.BlockSpec(shape, lambda i: (i,) + (0,) * (len(shape) - 1))

    out, bucket, sums, counts = pl.pallas_call(
        _fwd_body,
        grid=(GRID,),
        in_specs=[
            tiled((TILE, IN_DIM)),
            tiled((TILE, N_CLASSES)),
            tiled((TILE, LATENT_DIM)),
            full((IN_DIM, 512)), full((1, 512)),
            full((512, 1024)), full((1, 1024)),
            full((1024, 2 * LATENT_DIM)), full((1, 2 * LATENT_DIM)),
            full((LATENT_DIM, 1024)), full((N_CLASSES, 1024)), full((1, 1024)),
            full((1024, 512)), full((1, 512)),
            full((512, OUT_DIM)), full((1, OUT_DIM)),
            full((OUT_DIM, N_HASHES)), full((1, N_HASHES)),
            full((1, N_HASHES)),
        ],
        out_specs=[
            tiled((TILE, OUT_DIM)),
            tiled((TILE, 1)),
            full((NUM_BUCKETS, OUT_DIM)),
            full((NUM_BUCKETS, 1)),
        ],
        out_shape=[
            jax.ShapeDtypeStruct((BATCH, OUT_DIM), _f32),
            jax.ShapeDtypeStruct((BATCH, 1), jnp.int32),
            jax.ShapeDtypeStruct((NUM_BUCKETS, OUT_DIM), _f32),
            jax.ShapeDtypeStruct((NUM_BUCKETS, 1), _f32),
        ],
        interpret=interpret,
    )(images, labels, eps, We1, be1r, We2, be2r, We3, be3r,
      Wd1z, Wd1l, bd1r, Wd2, bd2r, Wd3, bd3r, a, br, primes)

    q = pl.pallas_call(
        _q_body,
        grid=(GRID,),
        in_specs=[
            tiled((TILE, OUT_DIM)),
            full((NUM_BUCKETS, OUT_DIM)),
            full((NUM_BUCKETS, 1)),
        ],
        out_specs=tiled((TILE, NUM_BUCKETS)),
        out_shape=jax.ShapeDtypeStruct((BATCH, NUM_BUCKETS), _f32),
        interpret=interpret,
    )(out, sums, counts)
    return q


def kernel(images, labels, We1, be1, We2, be2, We3, be3,
           Wd1, bd1, Wd2, bd2, Wd3, bd3, a, b, eps):
    return _run(images, labels, We1, be1, We2, be2, We3, be3,
                Wd1, bd1, Wd2, bd2, Wd3, bd3, a, b, eps)
